# baseline (speedup 1.0000x reference)
import functools

import jax
import jax.numpy as jnp
import numpy as np
from jax import lax
from jax.experimental import pallas as pl
from jax.experimental.pallas import tpu as pltpu
from jax.experimental.pallas import tpu_sc as plsc

N = 10000
E = 160000
F = 128
NRBF = 50
CUTOFF = 5.0
NI = 3
ZMAX = 100
NMOL = 16
LIMIT = 1.0
EPS = 1e-8

NPAD = 10240
EPAD = 163840
PW = 16
EB = 640
NB = 1024
EW = EPAD // 32
CF = 32
NF = EW // CF
CCH = 128
NCH = EW // CCH
RPS = NPAD // 16
COEFF = -0.5 / (CUTOFF / NRBF) ** 2
PI = float(np.pi)

_INTERPRET = False


def _ssp(x):
    return jnp.maximum(x, 0.0) + jnp.log1p(jnp.exp(-jnp.abs(x))) - np.log(2.0)


def _sig(x):
    e = jnp.exp(-jnp.abs(x))
    return jnp.where(x >= 0, 1.0 / (1.0 + e), e / (1.0 + e))


def _dot(a, b):
    return jnp.dot(a, b, preferred_element_type=jnp.float32)


_NSPEC = pl.BlockSpec((NB, F), lambda i: (i, 0))
_ESPEC = pl.BlockSpec((EB, F), lambda i: (i, 0))
_PSPEC = pl.BlockSpec((EB, PW), lambda i: (i, 0))
_MOLSPEC = pl.BlockSpec((1, 1, NB), lambda i: (i, 0, 0))
_DDSPEC = pl.BlockSpec((1, 1, EB), lambda i: (i, 0, 0))
_P2SPEC = pl.BlockSpec((2, NB, F), lambda i: (0, i, 0))


def _full(shape):
    n = len(shape)
    return pl.BlockSpec(shape, lambda i, _n=n: (0,) * _n)


def _embed_body(z_ref, emb_ref, win_ref, x0_ref, xw_ref):
    z = z_ref[0, 0, :]
    oh = (z[:, None] == lax.broadcasted_iota(jnp.int32, (NB, ZMAX), 1)).astype(jnp.float32)
    x0 = _dot(oh, emb_ref[...])
    rows = pl.program_id(0) * NB + lax.broadcasted_iota(jnp.int32, (NB, 1), 0)
    x0 = jnp.where(rows < N, x0, 0.0)
    x0_ref[...] = x0
    xw_ref[...] = _dot(x0, win_ref[...])


def _tc_embed(z3d, emb, win):
    return pl.pallas_call(
        _embed_body,
        grid=(NPAD // NB,),
        in_specs=[_MOLSPEC, _full((ZMAX, F)), _full((F, F))],
        out_specs=[_NSPEC, _NSPEC],
        out_shape=[jax.ShapeDtypeStruct((NPAD, F), jnp.float32)] * 2,
        interpret=_INTERPRET,
    )(z3d, emb, win)


def _centers_row():
    return (lax.broadcasted_iota(jnp.int32, (1, NRBF), 1).astype(jnp.float32)
            * (CUTOFF / (NRBF - 1)))


def _edge_geom(pj, pi):
    rij = pj - pi
    d = jnp.sqrt(jnp.sum(rij * rij, axis=1, keepdims=True) + 1e-12)
    centers = _centers_row()
    delta = d - centers
    rbf = jnp.exp(COEFF * delta * delta)
    inside = (d < CUTOFF).astype(jnp.float32)
    fcut = 0.5 * (jnp.cos(d * (PI / CUTOFF)) + 1.0) * inside
    return rij, d, rbf, fcut, inside


def _edge_mlp_body(pj_ref, pi_ref, wf1_ref, bf1_ref, wf2_ref, bf2_ref,
                   wij_ref):
    _, _, rbf, fcut, _ = _edge_geom(pj_ref[...], pi_ref[...])
    h1 = _dot(rbf, wf1_ref[...]) + bf1_ref[...]
    a1 = _ssp(h1)
    wij_ref[...] = (_dot(a1, wf2_ref[...]) + bf2_ref[...]) * fcut


def _tc_edge_mlp(pj, pi, wf1, bf1, wf2, bf2):
    return pl.pallas_call(
        _edge_mlp_body,
        grid=(EPAD // EB,),
        in_specs=[_PSPEC, _PSPEC, _full((NRBF, F)), _full((1, F)),
                  _full((F, F)), _full((1, F))],
        out_specs=[_ESPEC],
        out_shape=[jax.ShapeDtypeStruct((EPAD, F), jnp.float32)],
        interpret=_INTERPRET,
    )(pj, pi, wf1, bf1, wf2, bf2)[0]


def _node_body(aggp_ref, x_ref, w1_ref, b1_ref, w2_ref, b2_ref, winn_ref,
               agg_ref, xn_ref, xw_ref):
    agg = aggp_ref[0] + aggp_ref[1]
    agg_ref[...] = agg
    t = _dot(agg, w1_ref[...]) + b1_ref[...]
    v = _dot(_ssp(t), w2_ref[...]) + b2_ref[...]
    xn = x_ref[...] + v
    xn_ref[...] = xn
    xw_ref[...] = _dot(xn, winn_ref[...])


def _tc_node(aggp, x, w1, b1, w2, b2, winn):
    return pl.pallas_call(
        _node_body,
        grid=(NPAD // NB,),
        in_specs=[_P2SPEC, _NSPEC, _full((F, F)), _full((1, F)),
                  _full((F, F)), _full((1, F)), _full((F, F))],
        out_specs=[_NSPEC, _NSPEC, _NSPEC],
        out_shape=[jax.ShapeDtypeStruct((NPAD, F), jnp.float32)] * 3,
        interpret=_INTERPRET,
    )(aggp, x, w1, b1, w2, b2, winn)


def _node_last_body(aggp_ref, x_ref, w1_ref, b1_ref, w2_ref, b2_ref,
                    wa1_ref, ba1_ref, wa2_ref, ba2_ref, wa1t_ref, mol_ref,
                    agg_ref, emol_ref, dx_ref):
    agg = aggp_ref[0] + aggp_ref[1]
    agg_ref[...] = agg
    t = _dot(agg, w1_ref[...]) + b1_ref[...]
    v = _dot(_ssp(t), w2_ref[...]) + b2_ref[...]
    x3 = x_ref[...] + v
    y1 = _dot(x3, wa1_ref[...]) + ba1_ref[...]
    z = _ssp(y1)
    wa2 = wa2_ref[...]
    e_atom = jnp.sum(z * wa2, axis=1, keepdims=True) + ba2_ref[...]
    mol = mol_ref[0, 0, :]
    oh = (mol[:, None] == lax.broadcasted_iota(jnp.int32, (NB, 128), 1))
    part = jnp.sum(jnp.where(oh, e_atom, 0.0), axis=0, keepdims=True)

    @pl.when(pl.program_id(0) == 0)
    def _():
        emol_ref[...] = jnp.zeros_like(emol_ref)

    emol_ref[...] += part
    dy1 = wa2 * _sig(y1)
    dx_ref[...] = _dot(dy1, wa1t_ref[...])


def _tc_node_last(aggp, x, w1, b1, w2, b2, wa1, ba1, wa2r, ba2, wa1t, mol3d):
    return pl.pallas_call(
        _node_last_body,
        grid=(NPAD // NB,),
        in_specs=[_P2SPEC, _NSPEC, _full((F, F)), _full((1, F)),
                  _full((F, F)), _full((1, F)), _full((F, F // 2)),
                  _full((1, F // 2)), _full((1, F // 2)), _full((1, 1)),
                  _full((F // 2, F)), _MOLSPEC],
        out_specs=[_NSPEC, _full((1, 128)), _NSPEC],
        out_shape=[jax.ShapeDtypeStruct((NPAD, F), jnp.float32),
                   jax.ShapeDtypeStruct((1, 128), jnp.float32),
                   jax.ShapeDtypeStruct((NPAD, F), jnp.float32)],
        interpret=_INTERPRET,
    )(aggp, x, w1, b1, w2, b2, wa1, ba1, wa2r, ba2, wa1t, mol3d)


def _bwd_node_first_body(dx_ref, agg_ref, w1_ref, b1_ref, w2t_ref, w1t_ref,
                         dagg_ref):
    t = _dot(agg_ref[...], w1_ref[...]) + b1_ref[...]
    du = _dot(dx_ref[...], w2t_ref[...])
    dagg_ref[...] = _dot(du * _sig(t), w1t_ref[...])


def _tc_bwd_node_first(dx, agg, w1, b1, w2t, w1t):
    return pl.pallas_call(
        _bwd_node_first_body,
        grid=(NPAD // NB,),
        in_specs=[_NSPEC, _NSPEC, _full((F, F)), _full((1, F)),
                  _full((F, F)), _full((F, F))],
        out_specs=[_NSPEC],
        out_shape=[jax.ShapeDtypeStruct((NPAD, F), jnp.float32)],
        interpret=_INTERPRET,
    )(dx, agg, w1, b1, w2t, w1t)[0]


def _bwd_node_body(dxp_ref, dxwp_ref, wint_ref, agg_ref, w1_ref, b1_ref,
                   w2t_ref, w1t_ref, dx_ref, dagg_ref):
    dxw = dxwp_ref[0] + dxwp_ref[1]
    dx = dxp_ref[...] + _dot(dxw, wint_ref[...])
    dx_ref[...] = dx
    t = _dot(agg_ref[...], w1_ref[...]) + b1_ref[...]
    du = _dot(dx, w2t_ref[...])
    dagg_ref[...] = _dot(du * _sig(t), w1t_ref[...])


def _tc_bwd_node(dxp, dxwp, wint, agg, w1, b1, w2t, w1t):
    return pl.pallas_call(
        _bwd_node_body,
        grid=(NPAD // NB,),
        in_specs=[_NSPEC, _P2SPEC, _full((F, F)), _NSPEC, _full((F, F)),
                  _full((1, F)), _full((F, F)), _full((F, F))],
        out_specs=[_NSPEC, _NSPEC],
        out_shape=[jax.ShapeDtypeStruct((NPAD, F), jnp.float32)] * 2,
        interpret=_INTERPRET,
    )(dxp, dxwp, wint, agg, w1, b1, w2t, w1t)


def _edge_bwd_body(pj_ref, pi_ref, dw_ref, wf1_ref, bf1_ref, wf2_ref,
                   bf2_ref, wf2t_ref, wf1t_ref, dd_ref):
    _, d, rbf, fcut, inside = _edge_geom(pj_ref[...], pi_ref[...])
    centers = _centers_row()
    dfcut_dd = (-0.5 * PI / CUTOFF) * jnp.sin(d * (PI / CUTOFF)) * inside
    drbf_dd = rbf * (2.0 * COEFF) * (d - centers)
    h1 = _dot(rbf, wf1_ref[...]) + bf1_ref[...]
    a1 = _ssp(h1)
    wraw = _dot(a1, wf2_ref[...]) + bf2_ref[...]
    dwij = dw_ref[...]
    dwraw = dwij * fcut
    dfcut = jnp.sum(dwij * wraw, axis=1, keepdims=True)
    da1 = _dot(dwraw, wf2t_ref[...])
    dh1 = da1 * _sig(h1)
    drbf = _dot(dh1, wf1t_ref[...])
    dd = jnp.sum(drbf * drbf_dd, axis=1, keepdims=True) + dfcut * dfcut_dd
    dd_ref[...] = dd[:, 0][None, None, :]


def _tc_edge_bwd(pj, pi, dw, wf1, bf1, wf2, bf2, wf2t, wf1t):
    return pl.pallas_call(
        _edge_bwd_body,
        grid=(EPAD // EB,),
        in_specs=[_PSPEC, _PSPEC, _ESPEC, _full((NRBF, F)), _full((1, F)),
                  _full((F, F)), _full((1, F)), _full((F, F)),
                  _full((F, NRBF))],
        out_specs=[_DDSPEC],
        out_shape=[jax.ShapeDtypeStruct((EPAD // EB, 1, EB), jnp.float32)],
        interpret=_INTERPRET,
    )(pj, pi, dw, wf1, bf1, wf2, bf2, wf2t, wf1t)[0]


def _vec_body(pj_ref, pi_ref, d0_ref, d1_ref, d2_ref, vec_ref, nvec_ref):
    rij, d, _, _, _ = _edge_geom(pj_ref[...], pi_ref[...])
    dd = (d0_ref[0, 0, :] + d1_ref[0, 0, :] + d2_ref[0, 0, :])[:, None]
    vec = (dd / d) * rij
    vec_ref[...] = vec
    nvec_ref[...] = -vec


def _tc_vec(pj, pi, dd0, dd1, dd2):
    return pl.pallas_call(
        _vec_body,
        grid=(EPAD // EB,),
        in_specs=[_PSPEC, _PSPEC, _DDSPEC, _DDSPEC, _DDSPEC],
        out_specs=[_PSPEC, _PSPEC],
        out_shape=[jax.ShapeDtypeStruct((EPAD, PW), jnp.float32)] * 2,
        interpret=_INTERPRET,
    )(pj, pi, dd0, dd1, dd2)


def _norms_body(dposp_ref, mol_ref, f_ref, mm_ref):
    f = -(dposp_ref[0] + dposp_ref[1])
    f_ref[...] = f
    nrm = jnp.sqrt(jnp.sum(f * f, axis=1, keepdims=True))
    mol = mol_ref[0, 0, :]
    oh = (mol[:, None] == lax.broadcasted_iota(jnp.int32, (NB, 128), 1))
    masked = jnp.where(oh, nrm, -1.0)
    part = jnp.max(masked, axis=0, keepdims=True)

    @pl.when(pl.program_id(0) == 0)
    def _():
        mm_ref[...] = jnp.full_like(mm_ref, -1.0)

    mm_ref[...] = jnp.maximum(mm_ref[...], part)


def _tc_norms(dposp, mol3d):
    return pl.pallas_call(
        _norms_body,
        grid=(NPAD // NB,),
        in_specs=[pl.BlockSpec((2, NB, PW), lambda i: (0, i, 0)), _MOLSPEC],
        out_specs=[pl.BlockSpec((NB, PW), lambda i: (i, 0)),
                   _full((1, 128))],
        out_shape=[jax.ShapeDtypeStruct((NPAD, PW), jnp.float32),
                   jax.ShapeDtypeStruct((1, 128), jnp.float32)],
        interpret=_INTERPRET,
    )(dposp, mol3d)


def _action_body(f_ref, mm_ref, mol_ref, act_ref):
    mm = jnp.maximum(mm_ref[...], EPS)
    coef = jnp.minimum(LIMIT / mm, 1.0)
    mol = mol_ref[0, 0, :]
    oh = (mol[:, None] == lax.broadcasted_iota(jnp.int32, (NB, 128), 1))
    catom = jnp.sum(jnp.where(oh, coef, 0.0), axis=1, keepdims=True)
    act_ref[...] = f_ref[...] * catom


def _tc_action(forces, mm, mol3d):
    return pl.pallas_call(
        _action_body,
        grid=(NPAD // NB,),
        in_specs=[pl.BlockSpec((NB, PW), lambda i: (i, 0)),
                  _full((1, 128)), _MOLSPEC],
        out_specs=[pl.BlockSpec((NB, PW), lambda i: (i, 0))],
        out_shape=[jax.ShapeDtypeStruct((NPAD, PW), jnp.float32)],
        interpret=_INTERPRET,
    )(forces, mm, mol3d)[0]


def _sc_mesh():
    return plsc.VectorSubcoreMesh(core_axis_name="c", subcore_axis_name="s")


def _zero_vmem(buf, rows, width):
    def zrow(r, _):
        for k in range(width // 16):
            buf[r, pl.ds(k * 16, 16)] = jnp.zeros((16,), jnp.float32)
        return 0
    lax.fori_loop(0, rows, zrow, 0)


def _zero_shared(buf, shared, s, rows):
    for k in range(RPS // rows):
        pltpu.sync_copy(buf, shared.at[pl.ds(s * RPS + k * rows, rows)])


def _vcopy(dst, src, n):
    for k in range(n // 16):
        sl = pl.ds(k * 16, 16)
        dst[sl] = src[sl]


def sc_gather(table, idx):
    @functools.partial(
        pl.kernel,
        out_type=jax.ShapeDtypeStruct((EPAD, PW), jnp.float32),
        mesh=_sc_mesh(),
        compiler_params=pltpu.CompilerParams(use_tc_tiling_on_sc=False),
        scratch_types=[
            pltpu.VMEM((CCH,), jnp.int32),
            pltpu.VMEM((CCH,), jnp.int32),
            pltpu.VMEM((CCH, PW), jnp.float32),
            pltpu.VMEM((CCH, PW), jnp.float32),
            pltpu.SemaphoreType.DMA,
            pltpu.SemaphoreType.DMA,
            pltpu.SemaphoreType.DMA,
            pltpu.SemaphoreType.DMA,
            pltpu.SemaphoreType.DMA,
            pltpu.SemaphoreType.DMA,
        ],
    )
    def k(table_hbm, idx_hbm, out_hbm, i0, i1, r0, r1, si0, si1, sg0, sg1,
          so0, so1):
        wid = lax.axis_index("s") * 2 + lax.axis_index("c")
        base = wid * EW
        IV = (i0, i1)
        RV = (r0, r1)
        SI = (si0, si1)
        SG = (sg0, sg1)
        SO = (so0, so1)
        for b in range(2):
            pltpu.async_copy(idx_hbm.at[pl.ds(base + b * CCH, CCH)], IV[b], SI[b])

        def group(g, _):
            for half in range(2):
                c = 2 * g + half
                b = half
                bc = 1 - half

                @pl.when(c < NCH)
                def _():
                    pltpu.make_async_copy(idx_hbm.at[pl.ds(base, CCH)], IV[b], SI[b]).wait()

                    @pl.when(c >= 2)
                    def _():
                        pltpu.make_async_copy(RV[b], out_hbm.at[pl.ds(base, CCH)], SO[b]).wait()

                    pltpu.async_copy(table_hbm.at[IV[b]], RV[b], SG[b])

                @pl.when((c >= 1) & (c - 1 < NCH))
                def _():
                    cc = c - 1
                    pltpu.make_async_copy(table_hbm.at[IV[bc]], RV[bc], SG[bc]).wait()

                    @pl.when(cc + 2 < NCH)
                    def _():
                        pltpu.async_copy(
                            idx_hbm.at[pl.ds(base + (cc + 2) * CCH, CCH)],
                            IV[bc], SI[bc])

                    pltpu.async_copy(RV[bc], out_hbm.at[pl.ds(base + cc * CCH, CCH)], SO[bc])
            return 0

        lax.fori_loop(0, NCH // 2 + 1, group, 0)
        pltpu.make_async_copy(RV[0], out_hbm.at[pl.ds(base, CCH)], SO[0]).wait()
        pltpu.make_async_copy(RV[1], out_hbm.at[pl.ds(base, CCH)], SO[1]).wait()

    return k(table, idx)


def sc_conv_fwd(xw, wij, idxj, idxi):
    NBUF = 3
    L = NBUF - 1
    scr = []
    for _ in range(NBUF):
        scr += [pltpu.VMEM((CF,), jnp.int32), pltpu.VMEM((CF,), jnp.int32),
                pltpu.VMEM((CF,), jnp.int32),
                pltpu.VMEM((CF, F), jnp.float32),
                pltpu.VMEM((CF, F), jnp.float32)]
    scr.append(pltpu.VMEM_SHARED((NPAD, F), jnp.float32))
    scr += [pltpu.SemaphoreType.DMA] * (5 * NBUF)

    @functools.partial(
        pl.kernel,
        out_type=jax.ShapeDtypeStruct((2 * NPAD, F), jnp.float32),
        mesh=_sc_mesh(),
        scratch_types=scr,
    )
    def k(xw_hbm, wij_hbm, idxj_hbm, idxi_hbm, out_hbm, *s):
        slots = [s[5 * b:5 * b + 5] for b in range(NBUF)]
        agg_sh = s[5 * NBUF]
        sems = s[5 * NBUF + 1:]
        SIJ = sems[0:NBUF]
        SII = sems[NBUF:2 * NBUF]
        SG = sems[2 * NBUF:3 * NBUF]
        SW = sems[3 * NBUF:4 * NBUF]
        SS = sems[4 * NBUF:5 * NBUF]
        core = lax.axis_index("c")
        tid = lax.axis_index("s")
        wid = tid * 2 + core
        base = wid * EW
        _zero_vmem(slots[0][3], CF, F)
        _zero_shared(slots[0][3], agg_sh, tid, CF)
        plsc.subcore_barrier()
        for b in range(NBUF):
            off = base + b * CF
            pltpu.async_copy(idxj_hbm.at[pl.ds(off, CF)], slots[b][0], SIJ[b])
            pltpu.async_copy(idxi_hbm.at[pl.ds(off, CF)], slots[b][1], SII[b])

        def group(g, _):
            for half in range(NBUF):
                c = g * NBUF + half
                b = half
                ij, ii, isc, rows, wv = slots[b]

                @pl.when(c < NF)
                def _():
                    pltpu.make_async_copy(idxj_hbm.at[pl.ds(base, CF)], ij, SIJ[b]).wait()
                    pltpu.make_async_copy(idxi_hbm.at[pl.ds(base, CF)], ii, SII[b]).wait()

                    @pl.when(c >= NBUF)
                    def _():
                        pltpu.make_async_copy(rows, agg_sh.at[isc], SS[b]).wait()

                    off = base + c * CF
                    pltpu.async_copy(xw_hbm.at[ij], rows, SG[b])
                    pltpu.async_copy(wij_hbm.at[pl.ds(off, CF)], wv, SW[b])

                bc = (half - L) % NBUF
                ij2, ii2, isc2, rows2, wv2 = slots[bc]

                @pl.when((c >= L) & (c - L < NF))
                def _():
                    cc = c - L
                    pltpu.make_async_copy(xw_hbm.at[ij2], rows2, SG[bc]).wait()
                    pltpu.make_async_copy(wij_hbm.at[pl.ds(base, CF)], wv2, SW[bc]).wait()
                    _vcopy(isc2, ii2, CF)

                    @pl.when(cc + NBUF < NF)
                    def _():
                        off2 = base + (cc + NBUF) * CF
                        pltpu.async_copy(idxj_hbm.at[pl.ds(off2, CF)], ij2, SIJ[bc])
                        pltpu.async_copy(idxi_hbm.at[pl.ds(off2, CF)], ii2, SII[bc])

                    def mulrow(r, _2):
                        for kk in range(F // 16):
                            sl = pl.ds(kk * 16, 16)
                            rows2[r, sl] = rows2[r, sl] * wv2[r, sl]
                        return 0

                    lax.fori_loop(0, CF, mulrow, 0)
                    pltpu.async_copy(rows2, agg_sh.at[isc2], SS[bc], add=True)
            return 0

        lax.fori_loop(0, (NF + L + NBUF - 1) // NBUF + 1, group, 0)
        for b in range(NBUF):
            pltpu.make_async_copy(slots[b][3], agg_sh.at[slots[b][2]], SS[b]).wait()
        plsc.subcore_barrier()
        for k8 in range(RPS // CCH):
            r0 = tid * RPS + k8 * CCH
            pltpu.sync_copy(agg_sh.at[pl.ds(r0, CCH)],
                            out_hbm.at[pl.ds(core * NPAD + r0, CCH)])

    return k(xw, wij, idxj, idxi)


def sc_conv_bwd(dagg, xw, wij, idxj, idxi):
    NBUF = 2
    L = NBUF - 1
    scr = []
    for _ in range(NBUF):
        scr += [pltpu.VMEM((CF,), jnp.int32), pltpu.VMEM((CF,), jnp.int32),
                pltpu.VMEM((CF,), jnp.int32),
                pltpu.VMEM((CF, F), jnp.float32),
                pltpu.VMEM((CF, F), jnp.float32),
                pltpu.VMEM((CF, F), jnp.float32)]
    scr.append(pltpu.VMEM_SHARED((NPAD, F), jnp.float32))
    scr += [pltpu.SemaphoreType.DMA] * (7 * NBUF)

    @functools.partial(
        pl.kernel,
        out_type=(jax.ShapeDtypeStruct((2 * NPAD, F), jnp.float32),
                  jax.ShapeDtypeStruct((EPAD, F), jnp.float32)),
        mesh=_sc_mesh(),
        scratch_types=scr,
    )
    def k(dagg_hbm, xw_hbm, wij_hbm, idxj_hbm, idxi_hbm, dxw_hbm, dwij_hbm, *s):
        slots = [s[6 * b:6 * b + 6] for b in range(NBUF)]
        dxw_sh = s[6 * NBUF]
        sems = s[6 * NBUF + 1:]
        SIJ = sems[0:NBUF]
        SII = sems[NBUF:2 * NBUF]
        SG1 = sems[2 * NBUF:3 * NBUF]
        SG2 = sems[3 * NBUF:4 * NBUF]
        SW = sems[4 * NBUF:5 * NBUF]
        SS = sems[5 * NBUF:6 * NBUF]
        SD = sems[6 * NBUF:7 * NBUF]
        core = lax.axis_index("c")
        tid = lax.axis_index("s")
        wid = tid * 2 + core
        base = wid * EW
        _zero_vmem(slots[0][3], CF, F)
        _zero_shared(slots[0][3], dxw_sh, tid, CF)
        plsc.subcore_barrier()
        for b in range(NBUF):
            off = base + b * CF
            pltpu.async_copy(idxj_hbm.at[pl.ds(off, CF)], slots[b][0], SIJ[b])
            pltpu.async_copy(idxi_hbm.at[pl.ds(off, CF)], slots[b][1], SII[b])

        def group(g, _):
            for half in range(NBUF):
                c = g * NBUF + half
                b = half
                ij, ii, isc, dm, xwj, wv = slots[b]

                @pl.when(c < NF)
                def _():
                    pltpu.make_async_copy(idxj_hbm.at[pl.ds(base, CF)], ij, SIJ[b]).wait()
                    pltpu.make_async_copy(idxi_hbm.at[pl.ds(base, CF)], ii, SII[b]).wait()

                    @pl.when(c >= NBUF)
                    def _():
                        pltpu.make_async_copy(wv, dxw_sh.at[isc], SS[b]).wait()
                        pltpu.make_async_copy(xwj, dwij_hbm.at[pl.ds(base, CF)], SD[b]).wait()

                    off = base + c * CF
                    pltpu.async_copy(dagg_hbm.at[ii], dm, SG1[b])
                    pltpu.async_copy(xw_hbm.at[ij], xwj, SG2[b])
                    pltpu.async_copy(wij_hbm.at[pl.ds(off, CF)], wv, SW[b])

                bc = (half - L) % NBUF
                ij2, ii2, isc2, dm2, xwj2, wv2 = slots[bc]

                @pl.when((c >= L) & (c - L < NF))
                def _():
                    cc = c - L
                    pltpu.make_async_copy(dagg_hbm.at[ii2], dm2, SG1[bc]).wait()
                    pltpu.make_async_copy(xw_hbm.at[ij2], xwj2, SG2[bc]).wait()
                    pltpu.make_async_copy(wij_hbm.at[pl.ds(base, CF)], wv2, SW[bc]).wait()
                    _vcopy(isc2, ij2, CF)

                    @pl.when(cc + NBUF < NF)
                    def _():
                        off2 = base + (cc + NBUF) * CF
                        pltpu.async_copy(idxj_hbm.at[pl.ds(off2, CF)], ij2, SIJ[bc])
                        pltpu.async_copy(idxi_hbm.at[pl.ds(off2, CF)], ii2, SII[bc])

                    def mulrow(r, _2):
                        for kk in range(F // 16):
                            sl = pl.ds(kk * 16, 16)
                            a = dm2[r, sl]
                            xwj2[r, sl] = a * xwj2[r, sl]
                            wv2[r, sl] = a * wv2[r, sl]
                        return 0

                    lax.fori_loop(0, CF, mulrow, 0)
                    off3 = base + cc * CF
                    pltpu.async_copy(xwj2, dwij_hbm.at[pl.ds(off3, CF)], SD[bc])
                    pltpu.async_copy(wv2, dxw_sh.at[isc2], SS[bc], add=True)
            return 0

        lax.fori_loop(0, (NF + L + NBUF - 1) // NBUF + 1, group, 0)
        for b in range(NBUF):
            pltpu.make_async_copy(slots[b][5], dxw_sh.at[slots[b][2]], SS[b]).wait()
            pltpu.make_async_copy(slots[b][4], dwij_hbm.at[pl.ds(base, CF)], SD[b]).wait()
        plsc.subcore_barrier()
        for k8 in range(RPS // CCH):
            r0 = tid * RPS + k8 * CCH
            pltpu.sync_copy(dxw_sh.at[pl.ds(r0, CCH)],
                            dxw_hbm.at[pl.ds(core * NPAD + r0, CCH)])

    return k(dagg, xw, wij, idxj, idxi)


def sc_scatter_vec(vec, nvec, idxj, idxi):
    @functools.partial(
        pl.kernel,
        out_type=jax.ShapeDtypeStruct((2 * NPAD, PW), jnp.float32),
        mesh=_sc_mesh(),
        compiler_params=pltpu.CompilerParams(use_tc_tiling_on_sc=False),
        scratch_types=[
            pltpu.VMEM((CCH,), jnp.int32),
            pltpu.VMEM((CCH,), jnp.int32),
            pltpu.VMEM((CCH, PW), jnp.float32),
            pltpu.VMEM((CCH, PW), jnp.float32),
            pltpu.VMEM_SHARED((NPAD, PW), jnp.float32),
        ],
    )
    def k(vec_hbm, nvec_hbm, idxj_hbm, idxi_hbm, out_hbm,
          idxj_v, idxi_v, v_v, nv_v, dpos_sh):
        c = lax.axis_index("c")
        s = lax.axis_index("s")
        wid = s * 2 + c
        _zero_vmem(v_v, CCH, PW)
        _zero_shared(v_v, dpos_sh, s, CCH)
        plsc.subcore_barrier()

        def body(ch, _):
            off = wid * EW + ch * CCH
            pltpu.sync_copy(idxj_hbm.at[pl.ds(off, CCH)], idxj_v)
            pltpu.sync_copy(idxi_hbm.at[pl.ds(off, CCH)], idxi_v)
            pltpu.sync_copy(vec_hbm.at[pl.ds(off, CCH)], v_v)
            pltpu.sync_copy(nvec_hbm.at[pl.ds(off, CCH)], nv_v)
            pltpu.sync_copy(v_v, dpos_sh.at[idxj_v], add=True)
            pltpu.sync_copy(nv_v, dpos_sh.at[idxi_v], add=True)
            return 0

        lax.fori_loop(0, NCH, body, 0)
        plsc.subcore_barrier()
        for k8 in range(RPS // CCH):
            r0 = s * RPS + k8 * CCH
            pltpu.sync_copy(dpos_sh.at[pl.ds(r0, CCH)],
                            out_hbm.at[pl.ds(c * NPAD + r0, CCH)])

    return k(vec, nvec, idxj, idxi)


def kernel(positions, atomic_numbers, edge_index, mol_ids, params):
    f32 = jnp.float32
    pos_pad = jnp.zeros((NPAD, PW), f32).at[:N, :3].set(positions)
    z3d = (jnp.zeros((NPAD,), jnp.int32).at[:N].set(atomic_numbers)
           .reshape(NPAD // NB, 1, NB))
    idxi = jnp.full((EPAD,), N, jnp.int32).at[:E].set(edge_index[0])
    idxj = jnp.full((EPAD,), N, jnp.int32).at[:E].set(edge_index[1])
    mol3d = (jnp.full((NPAD,), NMOL, jnp.int32).at[:N].set(mol_ids)
             .reshape(NPAD // NB, 1, NB))
    ly = params['interactions']

    x0, xw0 = _tc_embed(z3d, params['embedding'], ly[0]['W_in'])
    pj = sc_gather(pos_pad, idxj)
    pi = sc_gather(pos_pad, idxi)
    wijs = [_tc_edge_mlp(pj, pi, ly[l]['Wf1'], ly[l]['bf1'][None, :],
                         ly[l]['Wf2'], ly[l]['bf2'][None, :])
            for l in range(NI)]

    xs, xws, aggs = [x0], [xw0], []
    for l in range(NI - 1):
        aggp = sc_conv_fwd(xws[l], wijs[l], idxj, idxi).reshape(2, NPAD, F)
        agg, xn, xwn = _tc_node(aggp, xs[l], ly[l]['W1'], ly[l]['b1'][None, :],
                                ly[l]['W2'], ly[l]['b2'][None, :],
                                ly[l + 1]['W_in'])
        aggs.append(agg)
        xs.append(xn)
        xws.append(xwn)

    aggp = sc_conv_fwd(xws[2], wijs[2], idxj, idxi).reshape(2, NPAD, F)
    aggL, emol128, dx3 = _tc_node_last(
        aggp, xs[2], ly[2]['W1'], ly[2]['b1'][None, :], ly[2]['W2'],
        ly[2]['b2'][None, :], params['Wa1'], params['ba1'][None, :],
        params['Wa2'][:, 0][None, :], params['ba2'][None, :],
        params['Wa1'].T, mol3d)
    aggs.append(aggL)

    dds = [None] * NI
    dagg = _tc_bwd_node_first(dx3, aggs[2], ly[2]['W1'],
                              ly[2]['b1'][None, :], ly[2]['W2'].T,
                              ly[2]['W1'].T)
    dxprev = dx3
    for l in range(NI - 1, 0, -1):
        dxwp, dwij = sc_conv_bwd(dagg, xws[l], wijs[l], idxj, idxi)
        dds[l] = _tc_edge_bwd(pj, pi, dwij, ly[l]['Wf1'],
                              ly[l]['bf1'][None, :], ly[l]['Wf2'],
                              ly[l]['bf2'][None, :], ly[l]['Wf2'].T,
                              ly[l]['Wf1'].T)
        dxn, dagg = _tc_bwd_node(dxprev, dxwp.reshape(2, NPAD, F),
                                 ly[l]['W_in'].T, aggs[l - 1],
                                 ly[l - 1]['W1'], ly[l - 1]['b1'][None, :],
                                 ly[l - 1]['W2'].T, ly[l - 1]['W1'].T)
        dxprev = dxn

    _, dwij = sc_conv_bwd(dagg, xws[0], wijs[0], idxj, idxi)
    dds[0] = _tc_edge_bwd(pj, pi, dwij, ly[0]['Wf1'],
                          ly[0]['bf1'][None, :], ly[0]['Wf2'],
                          ly[0]['bf2'][None, :], ly[0]['Wf2'].T,
                          ly[0]['Wf1'].T)

    vec, nvec = _tc_vec(pj, pi, dds[0], dds[1], dds[2])
    dposp = sc_scatter_vec(vec, nvec, idxj, idxi).reshape(2, NPAD, PW)
    forces, mm = _tc_norms(dposp, mol3d)
    act_pad = _tc_action(forces, mm, mol3d)

    return (act_pad[:N, :3], emol128[0, :NMOL])


# speedup vs baseline: 1.0078x; 1.0078x over previous
import functools

import jax
import jax.numpy as jnp
import numpy as np
from jax import lax
from jax.experimental import pallas as pl
from jax.experimental.pallas import tpu as pltpu
from jax.experimental.pallas import tpu_sc as plsc

N = 10000
E = 160000
F = 128
NRBF = 50
CUTOFF = 5.0
NI = 3
ZMAX = 100
NMOL = 16
LIMIT = 1.0
EPS = 1e-8

NPAD = 10240
EPAD = 163840
PW = 16
EB = 640
NB = 1024
EW = EPAD // 32
CF = 32
NF = EW // CF
CCH = 128
NCH = EW // CCH
RPS = NPAD // 16
COEFF = -0.5 / (CUTOFF / NRBF) ** 2
PI = float(np.pi)

_INTERPRET = False


def _ssp(x):
    return jnp.maximum(x, 0.0) + jnp.log1p(jnp.exp(-jnp.abs(x))) - np.log(2.0)


def _sig(x):
    e = jnp.exp(-jnp.abs(x))
    return jnp.where(x >= 0, 1.0 / (1.0 + e), e / (1.0 + e))


def _dot(a, b):
    return jnp.dot(a, b, preferred_element_type=jnp.float32)


_NSPEC = pl.BlockSpec((NB, F), lambda i: (i, 0))
_ESPEC = pl.BlockSpec((EB, F), lambda i: (i, 0))
_PSPEC = pl.BlockSpec((EB, PW), lambda i: (i, 0))
_MOLSPEC = pl.BlockSpec((1, 1, NB), lambda i: (i, 0, 0))
_DDSPEC = pl.BlockSpec((1, 1, EB), lambda i: (i, 0, 0))
_P2SPEC = pl.BlockSpec((2, NB, F), lambda i: (0, i, 0))


def _full(shape):
    n = len(shape)
    return pl.BlockSpec(shape, lambda i, _n=n: (0,) * _n)


def _embed_body(z_ref, emb_ref, win_ref, x0_ref, xw_ref):
    z = z_ref[0, 0, :]
    oh = (z[:, None] == lax.broadcasted_iota(jnp.int32, (NB, ZMAX), 1)).astype(jnp.float32)
    x0 = _dot(oh, emb_ref[...])
    rows = pl.program_id(0) * NB + lax.broadcasted_iota(jnp.int32, (NB, 1), 0)
    x0 = jnp.where(rows < N, x0, 0.0)
    x0_ref[...] = x0
    xw_ref[...] = _dot(x0, win_ref[...])


def _tc_embed(z3d, emb, win):
    return pl.pallas_call(
        _embed_body,
        grid=(NPAD // NB,),
        in_specs=[_MOLSPEC, _full((ZMAX, F)), _full((F, F))],
        out_specs=[_NSPEC, _NSPEC],
        out_shape=[jax.ShapeDtypeStruct((NPAD, F), jnp.float32)] * 2,
        interpret=_INTERPRET,
    )(z3d, emb, win)


def _centers_row():
    return (lax.broadcasted_iota(jnp.int32, (1, NRBF), 1).astype(jnp.float32)
            * (CUTOFF / (NRBF - 1)))


def _edge_geom(pj, pi):
    rij = pj - pi
    d = jnp.sqrt(jnp.sum(rij * rij, axis=1, keepdims=True) + 1e-12)
    centers = _centers_row()
    delta = d - centers
    rbf = jnp.exp(COEFF * delta * delta)
    inside = (d < CUTOFF).astype(jnp.float32)
    fcut = 0.5 * (jnp.cos(d * (PI / CUTOFF)) + 1.0) * inside
    return rij, d, rbf, fcut, inside


def _edge_mlp_body(pj_ref, pi_ref, wf1_ref, bf1_ref, wf2_ref, bf2_ref,
                   wij_ref, a1_ref, s1_ref):
    _, _, rbf, fcut, _ = _edge_geom(pj_ref[...], pi_ref[...])
    h1 = _dot(rbf, wf1_ref[...]) + bf1_ref[...]
    a1 = _ssp(h1)
    a1_ref[...] = a1
    s1_ref[...] = _sig(h1)
    wij_ref[...] = (_dot(a1, wf2_ref[...]) + bf2_ref[...]) * fcut


def _tc_edge_mlp(pj, pi, wf1, bf1, wf2, bf2):
    return pl.pallas_call(
        _edge_mlp_body,
        grid=(EPAD // EB,),
        in_specs=[_PSPEC, _PSPEC, _full((NRBF, F)), _full((1, F)),
                  _full((F, F)), _full((1, F))],
        out_specs=[_ESPEC, _ESPEC, _ESPEC],
        out_shape=[jax.ShapeDtypeStruct((EPAD, F), jnp.float32)] * 3,
        interpret=_INTERPRET,
    )(pj, pi, wf1, bf1, wf2, bf2)


def _node_body(aggp_ref, x_ref, w1_ref, b1_ref, w2_ref, b2_ref, winn_ref,
               agg_ref, xn_ref, xw_ref):
    agg = aggp_ref[0] + aggp_ref[1]
    agg_ref[...] = agg
    t = _dot(agg, w1_ref[...]) + b1_ref[...]
    v = _dot(_ssp(t), w2_ref[...]) + b2_ref[...]
    xn = x_ref[...] + v
    xn_ref[...] = xn
    xw_ref[...] = _dot(xn, winn_ref[...])


def _tc_node(aggp, x, w1, b1, w2, b2, winn):
    return pl.pallas_call(
        _node_body,
        grid=(NPAD // NB,),
        in_specs=[_P2SPEC, _NSPEC, _full((F, F)), _full((1, F)),
                  _full((F, F)), _full((1, F)), _full((F, F))],
        out_specs=[_NSPEC, _NSPEC, _NSPEC],
        out_shape=[jax.ShapeDtypeStruct((NPAD, F), jnp.float32)] * 3,
        interpret=_INTERPRET,
    )(aggp, x, w1, b1, w2, b2, winn)


def _node_last_body(aggp_ref, x_ref, w1_ref, b1_ref, w2_ref, b2_ref,
                    wa1_ref, ba1_ref, wa2_ref, ba2_ref, wa1t_ref, mol_ref,
                    agg_ref, emol_ref, dx_ref):
    agg = aggp_ref[0] + aggp_ref[1]
    agg_ref[...] = agg
    t = _dot(agg, w1_ref[...]) + b1_ref[...]
    v = _dot(_ssp(t), w2_ref[...]) + b2_ref[...]
    x3 = x_ref[...] + v
    y1 = _dot(x3, wa1_ref[...]) + ba1_ref[...]
    z = _ssp(y1)
    wa2 = wa2_ref[...]
    e_atom = jnp.sum(z * wa2, axis=1, keepdims=True) + ba2_ref[...]
    mol = mol_ref[0, 0, :]
    oh = (mol[:, None] == lax.broadcasted_iota(jnp.int32, (NB, 128), 1))
    part = jnp.sum(jnp.where(oh, e_atom, 0.0), axis=0, keepdims=True)

    @pl.when(pl.program_id(0) == 0)
    def _():
        emol_ref[...] = jnp.zeros_like(emol_ref)

    emol_ref[...] += part
    dy1 = wa2 * _sig(y1)
    dx_ref[...] = _dot(dy1, wa1t_ref[...])


def _tc_node_last(aggp, x, w1, b1, w2, b2, wa1, ba1, wa2r, ba2, wa1t, mol3d):
    return pl.pallas_call(
        _node_last_body,
        grid=(NPAD // NB,),
        in_specs=[_P2SPEC, _NSPEC, _full((F, F)), _full((1, F)),
                  _full((F, F)), _full((1, F)), _full((F, F // 2)),
                  _full((1, F // 2)), _full((1, F // 2)), _full((1, 1)),
                  _full((F // 2, F)), _MOLSPEC],
        out_specs=[_NSPEC, _full((1, 128)), _NSPEC],
        out_shape=[jax.ShapeDtypeStruct((NPAD, F), jnp.float32),
                   jax.ShapeDtypeStruct((1, 128), jnp.float32),
                   jax.ShapeDtypeStruct((NPAD, F), jnp.float32)],
        interpret=_INTERPRET,
    )(aggp, x, w1, b1, w2, b2, wa1, ba1, wa2r, ba2, wa1t, mol3d)


def _bwd_node_first_body(dx_ref, agg_ref, w1_ref, b1_ref, w2t_ref, w1t_ref,
                         dagg_ref):
    t = _dot(agg_ref[...], w1_ref[...]) + b1_ref[...]
    du = _dot(dx_ref[...], w2t_ref[...])
    dagg_ref[...] = _dot(du * _sig(t), w1t_ref[...])


def _tc_bwd_node_first(dx, agg, w1, b1, w2t, w1t):
    return pl.pallas_call(
        _bwd_node_first_body,
        grid=(NPAD // NB,),
        in_specs=[_NSPEC, _NSPEC, _full((F, F)), _full((1, F)),
                  _full((F, F)), _full((F, F))],
        out_specs=[_NSPEC],
        out_shape=[jax.ShapeDtypeStruct((NPAD, F), jnp.float32)],
        interpret=_INTERPRET,
    )(dx, agg, w1, b1, w2t, w1t)[0]


def _bwd_node_body(dxp_ref, dxwp_ref, wint_ref, agg_ref, w1_ref, b1_ref,
                   w2t_ref, w1t_ref, dx_ref, dagg_ref):
    dxw = dxwp_ref[0] + dxwp_ref[1]
    dx = dxp_ref[...] + _dot(dxw, wint_ref[...])
    dx_ref[...] = dx
    t = _dot(agg_ref[...], w1_ref[...]) + b1_ref[...]
    du = _dot(dx, w2t_ref[...])
    dagg_ref[...] = _dot(du * _sig(t), w1t_ref[...])


def _tc_bwd_node(dxp, dxwp, wint, agg, w1, b1, w2t, w1t):
    return pl.pallas_call(
        _bwd_node_body,
        grid=(NPAD // NB,),
        in_specs=[_NSPEC, _P2SPEC, _full((F, F)), _NSPEC, _full((F, F)),
                  _full((1, F)), _full((F, F)), _full((F, F))],
        out_specs=[_NSPEC, _NSPEC],
        out_shape=[jax.ShapeDtypeStruct((NPAD, F), jnp.float32)] * 2,
        interpret=_INTERPRET,
    )(dxp, dxwp, wint, agg, w1, b1, w2t, w1t)


def _edge_bwd_body(pj_ref, pi_ref, dw_ref, a1_ref, s1_ref, wf2_ref,
                   bf2_ref, wf2t_ref, wf1t_ref, dd_ref):
    _, d, rbf, fcut, inside = _edge_geom(pj_ref[...], pi_ref[...])
    centers = _centers_row()
    dfcut_dd = (-0.5 * PI / CUTOFF) * jnp.sin(d * (PI / CUTOFF)) * inside
    drbf_dd = rbf * (2.0 * COEFF) * (d - centers)
    a1 = a1_ref[...]
    wraw = _dot(a1, wf2_ref[...]) + bf2_ref[...]
    dwij = dw_ref[...]
    dwraw = dwij * fcut
    dfcut = jnp.sum(dwij * wraw, axis=1, keepdims=True)
    da1 = _dot(dwraw, wf2t_ref[...])
    dh1 = da1 * s1_ref[...]
    drbf = _dot(dh1, wf1t_ref[...])
    dd = jnp.sum(drbf * drbf_dd, axis=1, keepdims=True) + dfcut * dfcut_dd
    dd_ref[...] = dd[:, 0][None, None, :]


def _tc_edge_bwd(pj, pi, dw, a1, s1, wf2, bf2, wf2t, wf1t):
    return pl.pallas_call(
        _edge_bwd_body,
        grid=(EPAD // EB,),
        in_specs=[_PSPEC, _PSPEC, _ESPEC, _ESPEC, _ESPEC,
                  _full((F, F)), _full((1, F)), _full((F, F)),
                  _full((F, NRBF))],
        out_specs=[_DDSPEC],
        out_shape=[jax.ShapeDtypeStruct((EPAD // EB, 1, EB), jnp.float32)],
        interpret=_INTERPRET,
    )(pj, pi, dw, a1, s1, wf2, bf2, wf2t, wf1t)[0]


def _vec_body(pj_ref, pi_ref, d0_ref, d1_ref, d2_ref, vec_ref, nvec_ref):
    rij, d, _, _, _ = _edge_geom(pj_ref[...], pi_ref[...])
    dd = (d0_ref[0, 0, :] + d1_ref[0, 0, :] + d2_ref[0, 0, :])[:, None]
    vec = (dd / d) * rij
    vec_ref[...] = vec
    nvec_ref[...] = -vec


def _tc_vec(pj, pi, dd0, dd1, dd2):
    return pl.pallas_call(
        _vec_body,
        grid=(EPAD // EB,),
        in_specs=[_PSPEC, _PSPEC, _DDSPEC, _DDSPEC, _DDSPEC],
        out_specs=[_PSPEC, _PSPEC],
        out_shape=[jax.ShapeDtypeStruct((EPAD, PW), jnp.float32)] * 2,
        interpret=_INTERPRET,
    )(pj, pi, dd0, dd1, dd2)


def _norms_body(dposp_ref, mol_ref, f_ref, mm_ref):
    f = -(dposp_ref[0] + dposp_ref[1])
    f_ref[...] = f
    nrm = jnp.sqrt(jnp.sum(f * f, axis=1, keepdims=True))
    mol = mol_ref[0, 0, :]
    oh = (mol[:, None] == lax.broadcasted_iota(jnp.int32, (NB, 128), 1))
    masked = jnp.where(oh, nrm, -1.0)
    part = jnp.max(masked, axis=0, keepdims=True)

    @pl.when(pl.program_id(0) == 0)
    def _():
        mm_ref[...] = jnp.full_like(mm_ref, -1.0)

    mm_ref[...] = jnp.maximum(mm_ref[...], part)


def _tc_norms(dposp, mol3d):
    return pl.pallas_call(
        _norms_body,
        grid=(NPAD // NB,),
        in_specs=[pl.BlockSpec((2, NB, PW), lambda i: (0, i, 0)), _MOLSPEC],
        out_specs=[pl.BlockSpec((NB, PW), lambda i: (i, 0)),
                   _full((1, 128))],
        out_shape=[jax.ShapeDtypeStruct((NPAD, PW), jnp.float32),
                   jax.ShapeDtypeStruct((1, 128), jnp.float32)],
        interpret=_INTERPRET,
    )(dposp, mol3d)


def _action_body(f_ref, mm_ref, mol_ref, act_ref):
    mm = jnp.maximum(mm_ref[...], EPS)
    coef = jnp.minimum(LIMIT / mm, 1.0)
    mol = mol_ref[0, 0, :]
    oh = (mol[:, None] == lax.broadcasted_iota(jnp.int32, (NB, 128), 1))
    catom = jnp.sum(jnp.where(oh, coef, 0.0), axis=1, keepdims=True)
    act_ref[...] = f_ref[...] * catom


def _tc_action(forces, mm, mol3d):
    return pl.pallas_call(
        _action_body,
        grid=(NPAD // NB,),
        in_specs=[pl.BlockSpec((NB, PW), lambda i: (i, 0)),
                  _full((1, 128)), _MOLSPEC],
        out_specs=[pl.BlockSpec((NB, PW), lambda i: (i, 0))],
        out_shape=[jax.ShapeDtypeStruct((NPAD, PW), jnp.float32)],
        interpret=_INTERPRET,
    )(forces, mm, mol3d)[0]


def _sc_mesh():
    return plsc.VectorSubcoreMesh(core_axis_name="c", subcore_axis_name="s")


def _zero_vmem(buf, rows, width):
    def zrow(r, _):
        for k in range(width // 16):
            buf[r, pl.ds(k * 16, 16)] = jnp.zeros((16,), jnp.float32)
        return 0
    lax.fori_loop(0, rows, zrow, 0)


def _zero_shared(buf, shared, s, rows):
    for k in range(RPS // rows):
        pltpu.sync_copy(buf, shared.at[pl.ds(s * RPS + k * rows, rows)])


def _vcopy(dst, src, n):
    for k in range(n // 16):
        sl = pl.ds(k * 16, 16)
        dst[sl] = src[sl]


def sc_gather(table, idx):
    @functools.partial(
        pl.kernel,
        out_type=jax.ShapeDtypeStruct((EPAD, PW), jnp.float32),
        mesh=_sc_mesh(),
        compiler_params=pltpu.CompilerParams(use_tc_tiling_on_sc=False),
        scratch_types=[
            pltpu.VMEM((CCH,), jnp.int32),
            pltpu.VMEM((CCH,), jnp.int32),
            pltpu.VMEM((CCH, PW), jnp.float32),
            pltpu.VMEM((CCH, PW), jnp.float32),
            pltpu.SemaphoreType.DMA,
            pltpu.SemaphoreType.DMA,
            pltpu.SemaphoreType.DMA,
            pltpu.SemaphoreType.DMA,
            pltpu.SemaphoreType.DMA,
            pltpu.SemaphoreType.DMA,
        ],
    )
    def k(table_hbm, idx_hbm, out_hbm, i0, i1, r0, r1, si0, si1, sg0, sg1,
          so0, so1):
        wid = lax.axis_index("s") * 2 + lax.axis_index("c")
        base = wid * EW
        IV = (i0, i1)
        RV = (r0, r1)
        SI = (si0, si1)
        SG = (sg0, sg1)
        SO = (so0, so1)
        for b in range(2):
            pltpu.async_copy(idx_hbm.at[pl.ds(base + b * CCH, CCH)], IV[b], SI[b])

        def group(g, _):
            for half in range(2):
                c = 2 * g + half
                b = half
                bc = 1 - half

                @pl.when(c < NCH)
                def _():
                    pltpu.make_async_copy(idx_hbm.at[pl.ds(base, CCH)], IV[b], SI[b]).wait()

                    @pl.when(c >= 2)
                    def _():
                        pltpu.make_async_copy(RV[b], out_hbm.at[pl.ds(base, CCH)], SO[b]).wait()

                    pltpu.async_copy(table_hbm.at[IV[b]], RV[b], SG[b])

                @pl.when((c >= 1) & (c - 1 < NCH))
                def _():
                    cc = c - 1
                    pltpu.make_async_copy(table_hbm.at[IV[bc]], RV[bc], SG[bc]).wait()

                    @pl.when(cc + 2 < NCH)
                    def _():
                        pltpu.async_copy(
                            idx_hbm.at[pl.ds(base + (cc + 2) * CCH, CCH)],
                            IV[bc], SI[bc])

                    pltpu.async_copy(RV[bc], out_hbm.at[pl.ds(base + cc * CCH, CCH)], SO[bc])
            return 0

        lax.fori_loop(0, NCH // 2 + 1, group, 0)
        pltpu.make_async_copy(RV[0], out_hbm.at[pl.ds(base, CCH)], SO[0]).wait()
        pltpu.make_async_copy(RV[1], out_hbm.at[pl.ds(base, CCH)], SO[1]).wait()

    return k(table, idx)


def sc_conv_fwd(xw, wij, idxj, idxi):
    NBUF = 3
    L = NBUF - 1
    scr = []
    for _ in range(NBUF):
        scr += [pltpu.VMEM((CF,), jnp.int32), pltpu.VMEM((CF,), jnp.int32),
                pltpu.VMEM((CF,), jnp.int32),
                pltpu.VMEM((CF, F), jnp.float32),
                pltpu.VMEM((CF, F), jnp.float32)]
    scr.append(pltpu.VMEM_SHARED((NPAD, F), jnp.float32))
    scr += [pltpu.SemaphoreType.DMA] * (5 * NBUF)

    @functools.partial(
        pl.kernel,
        out_type=jax.ShapeDtypeStruct((2 * NPAD, F), jnp.float32),
        mesh=_sc_mesh(),
        scratch_types=scr,
    )
    def k(xw_hbm, wij_hbm, idxj_hbm, idxi_hbm, out_hbm, *s):
        slots = [s[5 * b:5 * b + 5] for b in range(NBUF)]
        agg_sh = s[5 * NBUF]
        sems = s[5 * NBUF + 1:]
        SIJ = sems[0:NBUF]
        SII = sems[NBUF:2 * NBUF]
        SG = sems[2 * NBUF:3 * NBUF]
        SW = sems[3 * NBUF:4 * NBUF]
        SS = sems[4 * NBUF:5 * NBUF]
        core = lax.axis_index("c")
        tid = lax.axis_index("s")
        wid = tid * 2 + core
        base = wid * EW
        _zero_vmem(slots[0][3], CF, F)
        _zero_shared(slots[0][3], agg_sh, tid, CF)
        plsc.subcore_barrier()
        for b in range(NBUF):
            off = base + b * CF
            pltpu.async_copy(idxj_hbm.at[pl.ds(off, CF)], slots[b][0], SIJ[b])
            pltpu.async_copy(idxi_hbm.at[pl.ds(off, CF)], slots[b][1], SII[b])

        def group(g, _):
            for half in range(NBUF):
                c = g * NBUF + half
                b = half
                ij, ii, isc, rows, wv = slots[b]

                @pl.when(c < NF)
                def _():
                    pltpu.make_async_copy(idxj_hbm.at[pl.ds(base, CF)], ij, SIJ[b]).wait()
                    pltpu.make_async_copy(idxi_hbm.at[pl.ds(base, CF)], ii, SII[b]).wait()

                    @pl.when(c >= NBUF)
                    def _():
                        pltpu.make_async_copy(rows, agg_sh.at[isc], SS[b]).wait()

                    off = base + c * CF
                    pltpu.async_copy(xw_hbm.at[ij], rows, SG[b])
                    pltpu.async_copy(wij_hbm.at[pl.ds(off, CF)], wv, SW[b])

                bc = (half - L) % NBUF
                ij2, ii2, isc2, rows2, wv2 = slots[bc]

                @pl.when((c >= L) & (c - L < NF))
                def _():
                    cc = c - L
                    pltpu.make_async_copy(xw_hbm.at[ij2], rows2, SG[bc]).wait()
                    pltpu.make_async_copy(wij_hbm.at[pl.ds(base, CF)], wv2, SW[bc]).wait()
                    _vcopy(isc2, ii2, CF)

                    @pl.when(cc + NBUF < NF)
                    def _():
                        off2 = base + (cc + NBUF) * CF
                        pltpu.async_copy(idxj_hbm.at[pl.ds(off2, CF)], ij2, SIJ[bc])
                        pltpu.async_copy(idxi_hbm.at[pl.ds(off2, CF)], ii2, SII[bc])

                    def mulrow(r, _2):
                        for kk in range(F // 16):
                            sl = pl.ds(kk * 16, 16)
                            rows2[r, sl] = rows2[r, sl] * wv2[r, sl]
                        return 0

                    lax.fori_loop(0, CF, mulrow, 0)
                    pltpu.async_copy(rows2, agg_sh.at[isc2], SS[bc], add=True)
            return 0

        lax.fori_loop(0, (NF + L + NBUF - 1) // NBUF + 1, group, 0)
        for b in range(NBUF):
            pltpu.make_async_copy(slots[b][3], agg_sh.at[slots[b][2]], SS[b]).wait()
        plsc.subcore_barrier()
        for k8 in range(RPS // CCH):
            r0 = tid * RPS + k8 * CCH
            pltpu.sync_copy(agg_sh.at[pl.ds(r0, CCH)],
                            out_hbm.at[pl.ds(core * NPAD + r0, CCH)])

    return k(xw, wij, idxj, idxi)


def sc_conv_bwd(dagg, xw, wij, idxj, idxi):
    NBUF = 2
    L = NBUF - 1
    scr = []
    for _ in range(NBUF):
        scr += [pltpu.VMEM((CF,), jnp.int32), pltpu.VMEM((CF,), jnp.int32),
                pltpu.VMEM((CF,), jnp.int32),
                pltpu.VMEM((CF, F), jnp.float32),
                pltpu.VMEM((CF, F), jnp.float32),
                pltpu.VMEM((CF, F), jnp.float32)]
    scr.append(pltpu.VMEM_SHARED((NPAD, F), jnp.float32))
    scr += [pltpu.SemaphoreType.DMA] * (7 * NBUF)

    @functools.partial(
        pl.kernel,
        out_type=(jax.ShapeDtypeStruct((2 * NPAD, F), jnp.float32),
                  jax.ShapeDtypeStruct((EPAD, F), jnp.float32)),
        mesh=_sc_mesh(),
        scratch_types=scr,
    )
    def k(dagg_hbm, xw_hbm, wij_hbm, idxj_hbm, idxi_hbm, dxw_hbm, dwij_hbm, *s):
        slots = [s[6 * b:6 * b + 6] for b in range(NBUF)]
        dxw_sh = s[6 * NBUF]
        sems = s[6 * NBUF + 1:]
        SIJ = sems[0:NBUF]
        SII = sems[NBUF:2 * NBUF]
        SG1 = sems[2 * NBUF:3 * NBUF]
        SG2 = sems[3 * NBUF:4 * NBUF]
        SW = sems[4 * NBUF:5 * NBUF]
        SS = sems[5 * NBUF:6 * NBUF]
        SD = sems[6 * NBUF:7 * NBUF]
        core = lax.axis_index("c")
        tid = lax.axis_index("s")
        wid = tid * 2 + core
        base = wid * EW
        _zero_vmem(slots[0][3], CF, F)
        _zero_shared(slots[0][3], dxw_sh, tid, CF)
        plsc.subcore_barrier()
        for b in range(NBUF):
            off = base + b * CF
            pltpu.async_copy(idxj_hbm.at[pl.ds(off, CF)], slots[b][0], SIJ[b])
            pltpu.async_copy(idxi_hbm.at[pl.ds(off, CF)], slots[b][1], SII[b])

        def group(g, _):
            for half in range(NBUF):
                c = g * NBUF + half
                b = half
                ij, ii, isc, dm, xwj, wv = slots[b]

                @pl.when(c < NF)
                def _():
                    pltpu.make_async_copy(idxj_hbm.at[pl.ds(base, CF)], ij, SIJ[b]).wait()
                    pltpu.make_async_copy(idxi_hbm.at[pl.ds(base, CF)], ii, SII[b]).wait()

                    @pl.when(c >= NBUF)
                    def _():
                        pltpu.make_async_copy(wv, dxw_sh.at[isc], SS[b]).wait()
                        pltpu.make_async_copy(xwj, dwij_hbm.at[pl.ds(base, CF)], SD[b]).wait()

                    off = base + c * CF
                    pltpu.async_copy(dagg_hbm.at[ii], dm, SG1[b])
                    pltpu.async_copy(xw_hbm.at[ij], xwj, SG2[b])
                    pltpu.async_copy(wij_hbm.at[pl.ds(off, CF)], wv, SW[b])

                bc = (half - L) % NBUF
                ij2, ii2, isc2, dm2, xwj2, wv2 = slots[bc]

                @pl.when((c >= L) & (c - L < NF))
                def _():
                    cc = c - L
                    pltpu.make_async_copy(dagg_hbm.at[ii2], dm2, SG1[bc]).wait()
                    pltpu.make_async_copy(xw_hbm.at[ij2], xwj2, SG2[bc]).wait()
                    pltpu.make_async_copy(wij_hbm.at[pl.ds(base, CF)], wv2, SW[bc]).wait()
                    _vcopy(isc2, ij2, CF)

                    @pl.when(cc + NBUF < NF)
                    def _():
                        off2 = base + (cc + NBUF) * CF
                        pltpu.async_copy(idxj_hbm.at[pl.ds(off2, CF)], ij2, SIJ[bc])
                        pltpu.async_copy(idxi_hbm.at[pl.ds(off2, CF)], ii2, SII[bc])

                    def mulrow(r, _2):
                        for kk in range(F // 16):
                            sl = pl.ds(kk * 16, 16)
                            a = dm2[r, sl]
                            xwj2[r, sl] = a * xwj2[r, sl]
                            wv2[r, sl] = a * wv2[r, sl]
                        return 0

                    lax.fori_loop(0, CF, mulrow, 0)
                    off3 = base + cc * CF
                    pltpu.async_copy(xwj2, dwij_hbm.at[pl.ds(off3, CF)], SD[bc])
                    pltpu.async_copy(wv2, dxw_sh.at[isc2], SS[bc], add=True)
            return 0

        lax.fori_loop(0, (NF + L + NBUF - 1) // NBUF + 1, group, 0)
        for b in range(NBUF):
            pltpu.make_async_copy(slots[b][5], dxw_sh.at[slots[b][2]], SS[b]).wait()
            pltpu.make_async_copy(slots[b][4], dwij_hbm.at[pl.ds(base, CF)], SD[b]).wait()
        plsc.subcore_barrier()
        for k8 in range(RPS // CCH):
            r0 = tid * RPS + k8 * CCH
            pltpu.sync_copy(dxw_sh.at[pl.ds(r0, CCH)],
                            dxw_hbm.at[pl.ds(core * NPAD + r0, CCH)])

    return k(dagg, xw, wij, idxj, idxi)


def sc_scatter_vec(vec, nvec, idxj, idxi):
    @functools.partial(
        pl.kernel,
        out_type=jax.ShapeDtypeStruct((2 * NPAD, PW), jnp.float32),
        mesh=_sc_mesh(),
        compiler_params=pltpu.CompilerParams(use_tc_tiling_on_sc=False),
        scratch_types=[
            pltpu.VMEM((CCH,), jnp.int32),
            pltpu.VMEM((CCH,), jnp.int32),
            pltpu.VMEM((CCH, PW), jnp.float32),
            pltpu.VMEM((CCH, PW), jnp.float32),
            pltpu.VMEM_SHARED((NPAD, PW), jnp.float32),
        ],
    )
    def k(vec_hbm, nvec_hbm, idxj_hbm, idxi_hbm, out_hbm,
          idxj_v, idxi_v, v_v, nv_v, dpos_sh):
        c = lax.axis_index("c")
        s = lax.axis_index("s")
        wid = s * 2 + c
        _zero_vmem(v_v, CCH, PW)
        _zero_shared(v_v, dpos_sh, s, CCH)
        plsc.subcore_barrier()

        def body(ch, _):
            off = wid * EW + ch * CCH
            pltpu.sync_copy(idxj_hbm.at[pl.ds(off, CCH)], idxj_v)
            pltpu.sync_copy(idxi_hbm.at[pl.ds(off, CCH)], idxi_v)
            pltpu.sync_copy(vec_hbm.at[pl.ds(off, CCH)], v_v)
            pltpu.sync_copy(nvec_hbm.at[pl.ds(off, CCH)], nv_v)
            pltpu.sync_copy(v_v, dpos_sh.at[idxj_v], add=True)
            pltpu.sync_copy(nv_v, dpos_sh.at[idxi_v], add=True)
            return 0

        lax.fori_loop(0, NCH, body, 0)
        plsc.subcore_barrier()
        for k8 in range(RPS // CCH):
            r0 = s * RPS + k8 * CCH
            pltpu.sync_copy(dpos_sh.at[pl.ds(r0, CCH)],
                            out_hbm.at[pl.ds(c * NPAD + r0, CCH)])

    return k(vec, nvec, idxj, idxi)


def kernel(positions, atomic_numbers, edge_index, mol_ids, params):
    f32 = jnp.float32
    pos_pad = jnp.zeros((NPAD, PW), f32).at[:N, :3].set(positions)
    z3d = (jnp.zeros((NPAD,), jnp.int32).at[:N].set(atomic_numbers)
           .reshape(NPAD // NB, 1, NB))
    idxi = jnp.full((EPAD,), N, jnp.int32).at[:E].set(edge_index[0])
    idxj = jnp.full((EPAD,), N, jnp.int32).at[:E].set(edge_index[1])
    mol3d = (jnp.full((NPAD,), NMOL, jnp.int32).at[:N].set(mol_ids)
             .reshape(NPAD // NB, 1, NB))
    ly = params['interactions']

    x0, xw0 = _tc_embed(z3d, params['embedding'], ly[0]['W_in'])
    pj = sc_gather(pos_pad, idxj)
    pi = sc_gather(pos_pad, idxi)
    emlp = [_tc_edge_mlp(pj, pi, ly[l]['Wf1'], ly[l]['bf1'][None, :],
                         ly[l]['Wf2'], ly[l]['bf2'][None, :])
            for l in range(NI)]
    wijs = [t[0] for t in emlp]

    xs, xws, aggs = [x0], [xw0], []
    for l in range(NI - 1):
        aggp = sc_conv_fwd(xws[l], wijs[l], idxj, idxi).reshape(2, NPAD, F)
        agg, xn, xwn = _tc_node(aggp, xs[l], ly[l]['W1'], ly[l]['b1'][None, :],
                                ly[l]['W2'], ly[l]['b2'][None, :],
                                ly[l + 1]['W_in'])
        aggs.append(agg)
        xs.append(xn)
        xws.append(xwn)

    aggp = sc_conv_fwd(xws[2], wijs[2], idxj, idxi).reshape(2, NPAD, F)
    aggL, emol128, dx3 = _tc_node_last(
        aggp, xs[2], ly[2]['W1'], ly[2]['b1'][None, :], ly[2]['W2'],
        ly[2]['b2'][None, :], params['Wa1'], params['ba1'][None, :],
        params['Wa2'][:, 0][None, :], params['ba2'][None, :],
        params['Wa1'].T, mol3d)
    aggs.append(aggL)

    dds = [None] * NI
    dagg = _tc_bwd_node_first(dx3, aggs[2], ly[2]['W1'],
                              ly[2]['b1'][None, :], ly[2]['W2'].T,
                              ly[2]['W1'].T)
    dxprev = dx3
    for l in range(NI - 1, 0, -1):
        dxwp, dwij = sc_conv_bwd(dagg, xws[l], wijs[l], idxj, idxi)
        dds[l] = _tc_edge_bwd(pj, pi, dwij, emlp[l][1], emlp[l][2],
                              ly[l]['Wf2'], ly[l]['bf2'][None, :],
                              ly[l]['Wf2'].T, ly[l]['Wf1'].T)
        dxn, dagg = _tc_bwd_node(dxprev, dxwp.reshape(2, NPAD, F),
                                 ly[l]['W_in'].T, aggs[l - 1],
                                 ly[l - 1]['W1'], ly[l - 1]['b1'][None, :],
                                 ly[l - 1]['W2'].T, ly[l - 1]['W1'].T)
        dxprev = dxn

    _, dwij = sc_conv_bwd(dagg, xws[0], wijs[0], idxj, idxi)
    dds[0] = _tc_edge_bwd(pj, pi, dwij, emlp[0][1], emlp[0][2],
                          ly[0]['Wf2'], ly[0]['bf2'][None, :],
                          ly[0]['Wf2'].T, ly[0]['Wf1'].T)

    vec, nvec = _tc_vec(pj, pi, dds[0], dds[1], dds[2])
    dposp = sc_scatter_vec(vec, nvec, idxj, idxi).reshape(2, NPAD, PW)
    forces, mm = _tc_norms(dposp, mol3d)
    act_pad = _tc_action(forces, mm, mol3d)

    return (act_pad[:N, :3], emol128[0, :NMOL])
